# baseline (speedup 1.0000x reference)
import functools

import jax
import jax.numpy as jnp
from jax import lax
from jax.experimental import pallas as pl
from jax.experimental.pallas import tpu as pltpu
from jax.experimental.pallas import tpu_sc as plsc

_N_CODES = 8192
_CODE_DIM = 256
_BETA = 0.25

_R = 1024
_K = 1024

_NC = 2
_NS = 16
_NW = _NC * _NS


def _assign(flat, cb):
    M, D = flat.shape
    N = cb.shape[0]
    gi, gk = M // _R, N // _K

    def body(x_ref, c_ref, idx_ref, commit_ref, mincmp, minval):
        i = pl.program_id(0)
        k = pl.program_id(1)
        x = x_ref[...]
        c = c_ref[...]
        fn = jnp.sum(x * x, axis=1, keepdims=True)
        cn = jnp.sum(c * c, axis=1)[None, :]

        def rtne_bf16(v):
            u = lax.bitcast_convert_type(v, jnp.uint32)
            u = u + jnp.uint32(0x7FFF) + ((u >> 16) & jnp.uint32(1))
            return lax.bitcast_convert_type(u & jnp.uint32(0xFFFF0000),
                                            jnp.float32)

        dot = lax.dot_general(rtne_bf16(x), rtne_bf16(c),
                              (((1,), (1,)), ((), ())),
                              preferred_element_type=jnp.float32)
        d = fn - 2.0 * dot + cn
        bmin = jnp.min(d, axis=1)
        cols = lax.broadcasted_iota(jnp.int32, (_R, _K), 1)
        barg = jnp.min(jnp.where(d == bmin[:, None], cols, N), axis=1) + k * _K

        @pl.when(k == 0)
        def _():
            mincmp[...] = bmin.astype(jnp.bfloat16).astype(jnp.float32)
            minval[...] = bmin
            idx_ref[...] = barg

        @pl.when(k != 0)
        def _():
            better = bmin < mincmp[...]
            nv = jnp.where(better, bmin, mincmp[...])
            mincmp[...] = nv.astype(jnp.bfloat16).astype(jnp.float32)
            minval[...] = jnp.where(better, bmin, minval[...])
            idx_ref[...] = jnp.where(better, barg, idx_ref[...])

        @pl.when(k == gk - 1)
        def _():
            @pl.when(i == 0)
            def _():
                commit_ref[...] = jnp.zeros((1, 1), jnp.float32)
            commit_ref[...] = commit_ref[...] + jnp.sum(minval[...])[None, None]

    return pl.pallas_call(
        body,
        grid=(gi, gk),
        in_specs=[pl.BlockSpec((_R, D), lambda i, k: (i, 0)),
                  pl.BlockSpec((_K, D), lambda i, k: (k, 0))],
        out_specs=[pl.BlockSpec((_R,), lambda i, k: (i,)),
                   pl.BlockSpec((1, 1), lambda i, k: (0, 0))],
        out_shape=[jax.ShapeDtypeStruct((M,), jnp.int32),
                   jax.ShapeDtypeStruct((1, 1), jnp.float32)],
        scratch_shapes=[pltpu.VMEM((_R,), jnp.float32),
                        pltpu.VMEM((_R,), jnp.float32)],
        compiler_params=pltpu.CompilerParams(
            dimension_semantics=("arbitrary", "arbitrary")),
    )(flat, cb)


def _gather_count(codebook, idx, zeros, ones):
    M = idx.shape[0]
    bpw = M // _NW
    ch = 128
    mesh = plsc.VectorSubcoreMesh(core_axis_name="c", subcore_axis_name="s")

    @functools.partial(
        pl.kernel,
        out_type=(jax.ShapeDtypeStruct((M, _CODE_DIM), jnp.float32),
                  jax.ShapeDtypeStruct((_NC, _N_CODES), jnp.float32)),
        mesh=mesh,
        scratch_types=[
            pltpu.VMEM((bpw,), jnp.int32),
            pltpu.VMEM((ch, _CODE_DIM), jnp.float32),
            pltpu.VMEM((bpw,), jnp.float32),
            pltpu.VMEM_SHARED((_N_CODES,), jnp.float32),
            pltpu.SemaphoreType.DMA,
        ],
    )
    def k(cb_hbm, idx_hbm, zeros_hbm, ones_hbm, zq_hbm, counts_hbm,
          idx_v, rows_v, ones_v, cshared, sem):
        cid = lax.axis_index("c")
        sid = lax.axis_index("s")
        wid = sid * _NC + cid
        base = wid * bpw
        pltpu.sync_copy(idx_hbm.at[pl.ds(base, bpw)], idx_v)

        @pl.when(sid == 0)
        def _():
            pltpu.sync_copy(zeros_hbm, cshared)
        plsc.subcore_barrier()
        pltpu.sync_copy(ones_hbm, ones_v)
        pltpu.sync_copy(ones_v, cshared.at[idx_v], add=True)

        for c in range(bpw // ch):
            pltpu.async_copy(cb_hbm.at[idx_v.at[pl.ds(c * ch, ch)]],
                             rows_v, sem).wait()
            pltpu.sync_copy(rows_v, zq_hbm.at[pl.ds(base + c * ch, ch)])

        plsc.subcore_barrier()

        @pl.when(sid == 0)
        def _():
            pltpu.sync_copy(cshared, counts_hbm.at[cid])

    return k(codebook, idx, zeros, ones)


def _ppl(counts2):
    def body(c_ref, out_ref):
        c = c_ref[0:1, :] + c_ref[1:2, :]
        s = jnp.sum(c)
        p = c / jnp.maximum(s, 1.0)
        safe = jnp.where(p > 0, p, 1.0)
        plogp = jnp.where(p > 0, p * jnp.log(safe), 0.0)
        out_ref[...] = jnp.exp(-jnp.sum(plogp))[None, None]

    return pl.pallas_call(
        body,
        out_shape=jax.ShapeDtypeStruct((1, 1), jnp.float32),
    )(counts2)


def kernel(z_e, codebook):
    B, T, D = z_e.shape
    flat = z_e.reshape(-1, D)
    M = flat.shape[0]
    idx, commit_raw = _assign(flat, codebook)
    zeros = jnp.zeros((_N_CODES,), jnp.float32)
    ones = jnp.ones((M // _NW,), jnp.float32)
    z_q, counts2 = _gather_count(codebook, idx, zeros, ones)
    ppl = _ppl(counts2)
    commit = commit_raw[0, 0] * (_BETA / (M * D))
    codebook_loss = jnp.zeros((), jnp.float32)
    return (z_q.reshape(B, T, D), commit.reshape(()), codebook_loss,
            ppl[0, 0].reshape(()), idx.reshape(B, T))
